# Initial kernel scaffold; baseline (speedup 1.0000x reference)
import jax
import jax.numpy as jnp
from jax.experimental import pallas as pl


def kernel(x, edge_index, edge_attr, u, batch, W, b):
    raise NotImplementedError("write your pallas kernel here")


# speedup vs baseline: 19.0013x; 19.0013x over previous
import functools

import jax
import jax.numpy as jnp
from jax import lax
from jax.experimental import pallas as pl
from jax.experimental.pallas import tpu as pltpu
from jax.experimental.pallas import tpu_sc as plsc

N_NODES = 50000
N_EDGES = 1600000
D_NODE = 128
D_EDGE = 16
N_GRAPHS = 512
D_U = 64
IN_DIM = D_NODE + D_EDGE + D_U
OUT_DIM = 128

NUM_CORES = 2
NUM_SUBCORES = 16
NUM_WORKERS = NUM_CORES * NUM_SUBCORES

EC = 512
E_CHUNKS = N_EDGES // EC
E_PER_W = E_CHUNKS // NUM_WORKERS
E_EXTRA = E_CHUNKS - E_PER_W * NUM_WORKERS

NCHUNK = 128
N_FULL_CHUNKS = N_NODES // NCHUNK
N_TAIL = N_NODES - N_FULL_CHUNKS * NCHUNK
N_PER_W = N_FULL_CHUNKS // NUM_WORKERS
N_EXTRA = N_FULL_CHUNKS - N_PER_W * NUM_WORKERS


def _agg_body(x_hbm, src_hbm, attr_hbm, batch_hbm,
              node_out, edge_out,
              batch_v, srcbuf, segbuf, attrbuf,
              nodebuf, nsegbuf, ntail_idx,
              nacc, eacc):
    c = lax.axis_index("c")
    s = lax.axis_index("s")
    w = c * NUM_SUBCORES + s

    zero16 = jnp.zeros((16,), jnp.float32)
    rows_n = N_GRAPHS // NUM_SUBCORES

    def _zn(i, _):
        r = i // 8
        q = i % 8
        nodebuf[r, pl.ds(q * 16, 16)] = zero16
        return _

    lax.fori_loop(0, rows_n * 8, _zn, 0)

    def _ze(i, _):
        attrbuf[i, :] = zero16
        return _

    lax.fori_loop(0, rows_n, _ze, 0)

    pltpu.sync_copy(nodebuf.at[pl.ds(0, rows_n)],
                    nacc.at[pl.ds(s * rows_n, rows_n)])
    pltpu.sync_copy(attrbuf.at[pl.ds(0, rows_n)],
                    eacc.at[pl.ds(s * rows_n, rows_n)])

    pltpu.sync_copy(batch_hbm, batch_v)
    plsc.subcore_barrier()

    n0 = w * N_PER_W + jnp.minimum(w, N_EXTRA)
    n_cnt = jnp.where(w < N_EXTRA, N_PER_W + 1, N_PER_W)

    def _node_chunk(j, _):
        base = j * NCHUNK
        pltpu.sync_copy(batch_hbm.at[pl.ds(base, NCHUNK)], nsegbuf)
        pltpu.sync_copy(x_hbm.at[pl.ds(base, NCHUNK)], nodebuf)
        pltpu.sync_copy(nodebuf, nacc.at[nsegbuf], add=True)
        return _

    lax.fori_loop(n0, n0 + n_cnt, _node_chunk, 0)

    @pl.when(w == NUM_WORKERS - 1)
    def _tail():
        base = N_FULL_CHUNKS * NCHUNK
        pltpu.sync_copy(batch_hbm.at[pl.ds(base, N_TAIL)], ntail_idx)
        pltpu.sync_copy(x_hbm.at[pl.ds(base, N_TAIL)],
                        nodebuf.at[pl.ds(0, N_TAIL)])
        pltpu.sync_copy(nodebuf.at[pl.ds(0, N_TAIL)],
                        nacc.at[ntail_idx], add=True)

    e0 = w * E_PER_W + jnp.minimum(w, E_EXTRA)
    e_cnt = jnp.where(w < E_EXTRA, E_PER_W + 1, E_PER_W)

    def _edge_chunk(e, _):
        base = e * EC
        pltpu.sync_copy(src_hbm.at[pl.ds(base, EC)], srcbuf)
        pltpu.sync_copy(attr_hbm.at[pl.ds(base, EC)], attrbuf)
        for k in range(EC // 16):
            idx16 = srcbuf[pl.ds(k * 16, 16)]
            seg16 = plsc.load_gather(batch_v, [idx16])
            segbuf[k // 8, pl.ds((k % 8) * 16, 16)] = seg16
        for j in range(EC // 128):
            pltpu.sync_copy(attrbuf.at[pl.ds(j * 128, 128)],
                            eacc.at[segbuf.at[j]], add=True)
        return _

    lax.fori_loop(e0, e0 + e_cnt, _edge_chunk, 0)

    plsc.subcore_barrier()

    @pl.when(s == 0)
    def _emit():
        pltpu.sync_copy(nacc, node_out.at[c])
        pltpu.sync_copy(eacc, edge_out.at[c])


_agg = functools.partial(
    pl.kernel,
    out_type=[
        jax.ShapeDtypeStruct((NUM_CORES, N_GRAPHS, D_NODE), jnp.float32),
        jax.ShapeDtypeStruct((NUM_CORES, N_GRAPHS, D_EDGE), jnp.float32),
    ],
    mesh=plsc.VectorSubcoreMesh(core_axis_name="c", subcore_axis_name="s"),
    compiler_params=pltpu.CompilerParams(
        needs_layout_passes=False, use_tc_tiling_on_sc=False),
    scratch_types=[
        pltpu.VMEM((N_NODES,), jnp.int32),
        pltpu.VMEM((EC,), jnp.int32),
        pltpu.VMEM((EC // 128, 128), jnp.int32),
        pltpu.VMEM((EC, D_EDGE), jnp.float32),
        pltpu.VMEM((NCHUNK, D_NODE), jnp.float32),
        pltpu.VMEM((NCHUNK,), jnp.int32),
        pltpu.VMEM((N_TAIL,), jnp.int32),
        pltpu.VMEM_SHARED((N_GRAPHS, D_NODE), jnp.float32),
        pltpu.VMEM_SHARED((N_GRAPHS, D_EDGE), jnp.float32),
    ],
)(_agg_body)


def _mlp_body(np_ref, ep_ref, u_ref, w_ref, b_ref, o_ref):
    n = np_ref[0] + np_ref[1]
    e = ep_ref[0] + ep_ref[1]
    w_n = w_ref[pl.ds(0, D_NODE), :]
    w_e = w_ref[pl.ds(D_NODE, D_EDGE), :]
    w_u = w_ref[pl.ds(D_NODE + D_EDGE, D_U), :]
    acc = jnp.dot(n, w_n, preferred_element_type=jnp.float32)
    acc += jnp.dot(e, w_e, preferred_element_type=jnp.float32)
    acc += jnp.dot(u_ref[...], w_u, preferred_element_type=jnp.float32)
    o_ref[...] = jnp.maximum(acc + b_ref[...], 0.0)


def _mlp(node_parts, edge_parts, u, W, b2d):
    return pl.pallas_call(
        _mlp_body,
        out_shape=jax.ShapeDtypeStruct((N_GRAPHS, OUT_DIM), jnp.float32),
    )(node_parts, edge_parts, u, W, b2d)


def kernel(x, edge_index, edge_attr, u, batch, W, b):
    src = edge_index[0].astype(jnp.int32)
    batch32 = batch.astype(jnp.int32)
    node_parts, edge_parts = _agg(x, src, edge_attr, batch32)
    return _mlp(node_parts, edge_parts, u, W, b.reshape(1, OUT_DIM))


# speedup vs baseline: 23.0399x; 1.2125x over previous
import functools

import jax
import jax.numpy as jnp
from jax import lax
from jax.experimental import pallas as pl
from jax.experimental.pallas import tpu as pltpu
from jax.experimental.pallas import tpu_sc as plsc

N_NODES = 50000
N_EDGES = 1600000
D_NODE = 128
D_EDGE = 16
N_GRAPHS = 512
D_U = 64
IN_DIM = D_NODE + D_EDGE + D_U
OUT_DIM = 128

NUM_CORES = 2
NUM_SUBCORES = 16
NUM_WORKERS = NUM_CORES * NUM_SUBCORES

EC = 1280
E_CHUNKS = N_EDGES // EC
E_PER_W = E_CHUNKS // NUM_WORKERS
E_EXTRA = E_CHUNKS - E_PER_W * NUM_WORKERS
NSUB = EC // 128
NGRP = EC // 16

NCHUNK = 128
N_FULL_CHUNKS = N_NODES // NCHUNK
N_TAIL = N_NODES - N_FULL_CHUNKS * NCHUNK
N_PER_W = N_FULL_CHUNKS // NUM_WORKERS
N_EXTRA = N_FULL_CHUNKS - N_PER_W * NUM_WORKERS


def _agg_body(x_hbm, src_hbm, attr_hbm, batch_hbm,
              node_out, edge_out,
              batch_v, srcbuf0, srcbuf1, segbuf0, segbuf1,
              attrbuf0, attrbuf1,
              nodebuf, nsegbuf, ntail_idx,
              sem_in0, sem_in1, sem_sc0, sem_sc1,
              nacc, eacc):
    c = lax.axis_index("c")
    s = lax.axis_index("s")
    w = c * NUM_SUBCORES + s

    zero16 = jnp.zeros((16,), jnp.float32)
    rows_n = N_GRAPHS // NUM_SUBCORES

    def _zn(i, carry):
        nodebuf[i // 8, pl.ds((i % 8) * 16, 16)] = zero16
        return carry

    lax.fori_loop(0, rows_n * 8, _zn, 0)

    def _ze(i, carry):
        attrbuf0[i, :] = zero16
        return carry

    lax.fori_loop(0, rows_n, _ze, 0)

    pltpu.sync_copy(nodebuf.at[pl.ds(0, rows_n)],
                    nacc.at[pl.ds(s * rows_n, rows_n)])
    pltpu.sync_copy(attrbuf0.at[pl.ds(0, rows_n)],
                    eacc.at[pl.ds(s * rows_n, rows_n)])

    pltpu.sync_copy(batch_hbm, batch_v)
    plsc.subcore_barrier()

    n0 = w * N_PER_W + jnp.minimum(w, N_EXTRA)
    n_cnt = jnp.where(w < N_EXTRA, N_PER_W + 1, N_PER_W)

    def _node_chunk(j, carry):
        base = j * NCHUNK
        pltpu.sync_copy(batch_hbm.at[pl.ds(base, NCHUNK)], nsegbuf)
        pltpu.sync_copy(x_hbm.at[pl.ds(base, NCHUNK)], nodebuf)
        pltpu.sync_copy(nodebuf, nacc.at[nsegbuf], add=True)
        return carry

    lax.fori_loop(n0, n0 + n_cnt, _node_chunk, 0)

    @pl.when(w == NUM_WORKERS - 1)
    def _tail():
        base = N_FULL_CHUNKS * NCHUNK
        pltpu.sync_copy(batch_hbm.at[pl.ds(base, N_TAIL)], ntail_idx)
        pltpu.sync_copy(x_hbm.at[pl.ds(base, N_TAIL)],
                        nodebuf.at[pl.ds(0, N_TAIL)])
        pltpu.sync_copy(nodebuf.at[pl.ds(0, N_TAIL)],
                        nacc.at[ntail_idx], add=True)

    e0 = w * E_PER_W + jnp.minimum(w, E_EXTRA)
    e_cnt = jnp.where(w < E_EXTRA, E_PER_W + 1, E_PER_W)

    bufs = ((srcbuf0, segbuf0, attrbuf0, sem_in0, sem_sc0),
            (srcbuf1, segbuf1, attrbuf1, sem_in1, sem_sc1))

    def _issue_in(cid, srcb, attrb, semi):
        base = cid * EC
        pltpu.async_copy(src_hbm.at[pl.ds(base, EC)], srcb, semi)
        pltpu.async_copy(attr_hbm.at[pl.ds(base, EC)], attrb, semi)

    def _wait_in(cid, srcb, attrb, semi):
        base = cid * EC
        pltpu.make_async_copy(src_hbm.at[pl.ds(base, EC)], srcb, semi).wait()
        pltpu.make_async_copy(attr_hbm.at[pl.ds(base, EC)], attrb, semi).wait()

    def _gather(srcb, segb):
        for k in range(NGRP):
            idx16 = srcb[pl.ds(k * 16, 16)]
            seg16 = plsc.load_gather(batch_v, [idx16])
            segb[k // 8, pl.ds((k % 8) * 16, 16)] = seg16

    def _fire_scatter(attrb, segb, semsc):
        for j in range(NSUB):
            pltpu.async_copy(attrb.at[pl.ds(j * 128, 128)],
                             eacc.at[segb.at[j]], semsc, add=True)

    def _drain_scatter(attrb, segb, semsc):
        for j in range(NSUB):
            pltpu.make_async_copy(attrb.at[pl.ds(j * 128, 128)],
                                  eacc.at[segb.at[j]], semsc).wait()

    _issue_in(e0, srcbuf0, attrbuf0, sem_in0)

    def _outer(t, carry):
        for b in (0, 1):
            rel = t * 2 + b
            cid = e0 + rel
            srcb, segb, attrb, semi, semsc = bufs[b]
            osrcb, osegb, oattrb, osemi, osemsc = bufs[1 - b]

            @pl.when(rel < e_cnt)
            def _process():
                @pl.when(rel >= 1)
                def _():
                    _drain_scatter(oattrb, osegb, osemsc)

                @pl.when(rel + 1 < e_cnt)
                def _():
                    _issue_in(cid + 1, osrcb, oattrb, osemi)

                _wait_in(cid, srcb, attrb, semi)
                _gather(srcb, segb)
                _fire_scatter(attrb, segb, semsc)
        return carry

    lax.fori_loop(0, (e_cnt + 1) // 2, _outer, 0)

    @pl.when((e_cnt - 1) % 2 == 0)
    def _():
        _drain_scatter(attrbuf0, segbuf0, sem_sc0)

    @pl.when((e_cnt - 1) % 2 == 1)
    def _():
        _drain_scatter(attrbuf1, segbuf1, sem_sc1)

    plsc.subcore_barrier()

    @pl.when(s == 0)
    def _emit():
        pltpu.sync_copy(nacc, node_out.at[c])
        pltpu.sync_copy(eacc, edge_out.at[c])


_agg = functools.partial(
    pl.kernel,
    out_type=[
        jax.ShapeDtypeStruct((NUM_CORES, N_GRAPHS, D_NODE), jnp.float32),
        jax.ShapeDtypeStruct((NUM_CORES, N_GRAPHS, D_EDGE), jnp.float32),
    ],
    mesh=plsc.VectorSubcoreMesh(core_axis_name="c", subcore_axis_name="s"),
    compiler_params=pltpu.CompilerParams(
        needs_layout_passes=False, use_tc_tiling_on_sc=False),
    scratch_types=[
        pltpu.VMEM((N_NODES,), jnp.int32),
        pltpu.VMEM((EC,), jnp.int32),
        pltpu.VMEM((EC,), jnp.int32),
        pltpu.VMEM((NSUB, 128), jnp.int32),
        pltpu.VMEM((NSUB, 128), jnp.int32),
        pltpu.VMEM((EC, D_EDGE), jnp.float32),
        pltpu.VMEM((EC, D_EDGE), jnp.float32),
        pltpu.VMEM((NCHUNK, D_NODE), jnp.float32),
        pltpu.VMEM((NCHUNK,), jnp.int32),
        pltpu.VMEM((N_TAIL,), jnp.int32),
        pltpu.SemaphoreType.DMA,
        pltpu.SemaphoreType.DMA,
        pltpu.SemaphoreType.DMA,
        pltpu.SemaphoreType.DMA,
        pltpu.VMEM_SHARED((N_GRAPHS, D_NODE), jnp.float32),
        pltpu.VMEM_SHARED((N_GRAPHS, D_EDGE), jnp.float32),
    ],
)(_agg_body)


def _mlp_body(np_ref, ep_ref, u_ref, w_ref, b_ref, o_ref):
    n = np_ref[0] + np_ref[1]
    e = ep_ref[0] + ep_ref[1]
    w_n = w_ref[pl.ds(0, D_NODE), :]
    w_e = w_ref[pl.ds(D_NODE, D_EDGE), :]
    w_u = w_ref[pl.ds(D_NODE + D_EDGE, D_U), :]
    acc = jnp.dot(n, w_n, preferred_element_type=jnp.float32)
    acc += jnp.dot(e, w_e, preferred_element_type=jnp.float32)
    acc += jnp.dot(u_ref[...], w_u, preferred_element_type=jnp.float32)
    o_ref[...] = jnp.maximum(acc + b_ref[...], 0.0)


def _mlp(node_parts, edge_parts, u, W, b2d):
    return pl.pallas_call(
        _mlp_body,
        out_shape=jax.ShapeDtypeStruct((N_GRAPHS, OUT_DIM), jnp.float32),
    )(node_parts, edge_parts, u, W, b2d)


def kernel(x, edge_index, edge_attr, u, batch, W, b):
    src = edge_index[0].astype(jnp.int32)
    batch32 = batch.astype(jnp.int32)
    node_parts, edge_parts = _agg(x, src, edge_attr, batch32)
    return _mlp(node_parts, edge_parts, u, W, b.reshape(1, OUT_DIM))
